# Initial kernel scaffold; baseline (speedup 1.0000x reference)
import jax
import jax.numpy as jnp
from jax.experimental import pallas as pl


def kernel(x, edge_weight, conv_w, conv_b, gcn_w, gcn_b, press_w, press_b, jk_w0, jk_b0, jk_w1, jk_b1, jk_w2, jk_b2, fc1_w, fc1_b, fc2_w, fc2_b, bn1_g, bn1_b, bn2_g, bn2_b, bn3_g, bn3_b, edge_index):
    raise NotImplementedError("write your pallas kernel here")


# speedup vs baseline: 5.0657x; 5.0657x over previous
import functools

import jax
import jax.numpy as jnp
from jax import lax
from jax.experimental import pallas as pl
from jax.experimental.pallas import tpu as pltpu
from jax.experimental.pallas import tpu_sc as plsc

N = 10000
NPAD = 10240
E = 320000
NCORES = 2
NSUB = 16
NW = NCORES * NSUB
CHUNK = 128
NCPT = (E + NW * CHUNK - 1) // (NW * CHUNK)
EPAD = NW * CHUNK * NCPT
BN_SCALE = float(1.0 / (1.0 + 1e-5) ** 0.5)
F32 = jnp.float32


def _shift_rows(a, off, rows):
    if off == 0:
        return a
    z = jnp.zeros((abs(off), a.shape[1]), a.dtype)
    if off > 0:
        return jnp.concatenate([a[off:], z], axis=0)
    return jnp.concatenate([z, a[:off]], axis=0)


def _shift_cols(a, off):
    if off == 0:
        return a
    z = jnp.zeros((a.shape[0], abs(off)), a.dtype)
    if off > 0:
        return jnp.concatenate([a[:, off:], z], axis=1)
    return jnp.concatenate([z, a[:, :off]], axis=1)


def _dot(a, b):
    return jnp.dot(a, b, preferred_element_type=F32,
                   precision=jax.lax.Precision.HIGHEST)


BN_CONV = 2048
PXROWS = 3 + N + (NPAD + 6 - N - 3)


def _conv_body(px_ref, cw_ref, cb_ref, bng_ref, bnb_ref, gw_ref,
               taba_ref, tabb_ref):
    i = pl.program_id(0)
    base = i * BN_CONV
    xs = {off: px_ref[pl.ds(base + 3 + off, BN_CONV), :]
          for off in range(-3, 4)}
    h0 = []
    for b in range(3):
        d = b + 1
        y = jnp.broadcast_to(cb_ref[b][None, :], (BN_CONV, 96)).astype(F32)
        for k in range(3):
            y = y + _dot(xs[(k - 1) * d], cw_ref[b, :, :, k].T)
        h0.append(y)
    scale = bng_ref[...] * BN_SCALE
    parts = []
    for b in range(3):
        h1 = (jax.nn.relu(h0[b]) + h0[(b + 1) % 3]) * scale + bnb_ref[...]
        parts.append(_dot(h1, gw_ref[b]))
    tab = jnp.concatenate(parts, axis=1)
    taba_ref[...] = tab[:, :96]
    tabb_ref[...] = jnp.concatenate(
        [tab[:, 96:], jnp.zeros((BN_CONV, 80 - 66), F32)], axis=1)


def _conv_call(x, conv_w, conv_b, bn1_g, bn1_b, gcn_w):
    px = jnp.concatenate(
        [jnp.zeros((3, 128), F32), x, jnp.zeros((PXROWS - 3 - N, 128), F32)])
    full = lambda shape: pl.BlockSpec(shape, lambda i: (0,) * len(shape))
    return pl.pallas_call(
        _conv_body,
        grid=(NPAD // BN_CONV,),
        in_specs=[full((PXROWS, 128)), full((3, 96, 128, 3)), full((3, 96)),
                  full((1, 96)), full((1, 96)), full((3, 96, 54))],
        out_specs=(pl.BlockSpec((BN_CONV, 96), lambda i: (i, 0)),
                   pl.BlockSpec((BN_CONV, 80), lambda i: (i, 0))),
        out_shape=(jax.ShapeDtypeStruct((NPAD, 96), F32),
                   jax.ShapeDtypeStruct((NPAD, 80), F32)),
    )(px, conv_w, conv_b, bn1_g, bn1_b, gcn_w)


_GDN = lax.GatherDimensionNumbers(
    offset_dims=(), collapsed_slice_dims=(0,), start_index_map=(0,))


def _bcast_lane(vec, i):
    idx = jnp.full((16, 1), i, jnp.int32)
    return lax.gather(vec, idx, _GDN, slice_sizes=(1,),
                      mode=lax.GatherScatterMode.PROMISE_IN_BOUNDS)


def _pass_body(dpad, tab_h, src_h, dst_h, ew_h, out_h,
               src_v, dst_v, ew_v, rows_v, acc, sem):
    c = lax.axis_index("c")
    s = lax.axis_index("s")
    wid = c * NSUB + s
    nj = dpad // 16

    def zero_row(r, carry):
        for j in range(nj):
            rows_v[r, pl.ds(16 * j, 16)] = jnp.zeros((16,), F32)
        return carry

    lax.fori_loop(0, CHUNK, zero_row, 0)
    for q in range(NPAD // CHUNK // NSUB):
        blk = s * (NPAD // CHUNK // NSUB) + q
        pltpu.sync_copy(rows_v, acc.at[pl.ds(blk * CHUNK, CHUNK)])
    plsc.subcore_barrier()

    def chunk_body(ci, carry):
        base = (wid * NCPT + ci) * CHUNK
        pltpu.sync_copy(src_h.at[pl.ds(base, CHUNK)], src_v)
        pltpu.sync_copy(dst_h.at[pl.ds(base, CHUNK)], dst_v)
        pltpu.sync_copy(ew_h.at[pl.ds(base, CHUNK)], ew_v)
        pltpu.async_copy(tab_h.at[src_v], rows_v, sem).wait()

        def group_body(g, carry2):
            ewg = ew_v[pl.ds(g * 16, 16)]
            for i in range(16):
                w = _bcast_lane(ewg, i)
                e = g * 16 + i
                for j in range(nj):
                    rows_v[e, pl.ds(16 * j, 16)] = (
                        rows_v[e, pl.ds(16 * j, 16)] * w)
            return carry2

        lax.fori_loop(0, CHUNK // 16, group_body, 0)
        pltpu.sync_copy(rows_v, acc.at[dst_v], add=True)
        return carry

    lax.fori_loop(0, NCPT, chunk_body, 0)
    plsc.subcore_barrier()

    rows_per = NPAD // NSUB
    pltpu.sync_copy(acc.at[pl.ds(s * rows_per, rows_per)],
                    out_h.at[c, pl.ds(s * rows_per, rows_per)])


def _seg_pass(table, src, dst, ew, dpad):
    mesh = plsc.VectorSubcoreMesh(core_axis_name="c", subcore_axis_name="s")
    f = pl.kernel(
        functools.partial(_pass_body, dpad),
        mesh=mesh,
        compiler_params=pltpu.CompilerParams(use_tc_tiling_on_sc=False),
        out_type=jax.ShapeDtypeStruct((2, NPAD, dpad), F32),
        scratch_types=[
            pltpu.VMEM((CHUNK,), jnp.int32),
            pltpu.VMEM((CHUNK,), jnp.int32),
            pltpu.VMEM((CHUNK,), F32),
            pltpu.VMEM((CHUNK, dpad), F32),
            pltpu.VMEM_SHARED((NPAD, dpad), F32),
            pltpu.SemaphoreType.DMA,
        ],
    )
    return f(table, src, dst, ew)


def _mid_body(pa_ref, pb_ref, gb_ref, pw_ref, pb2_ref, bng_ref, bnb_ref,
              w0_ref, emd_ref, tab_ref):
    agg_a = pa_ref[0] + pa_ref[1]
    agg_b = pb_ref[0, :, :66] + pb_ref[1, :, :66]
    agg = jnp.concatenate([agg_a, agg_b], axis=1)
    h2 = [jax.nn.relu(agg[:, 54 * i:54 * (i + 1)] + gb_ref[i][None, :])
          for i in range(3)]
    pressed = jnp.full((BN_CONV, 54), pb2_ref[0, 0], F32)
    for cidx in range(3):
        for k in range(3):
            pressed = pressed + pw_ref[0, cidx, k] * _shift_cols(h2[cidx], k - 1)
    emd = pressed * (bng_ref[...] * BN_SCALE) + bnb_ref[...]
    emd_ref[...] = emd
    t = _dot(emd, w0_ref[...])
    tab_ref[...] = jnp.concatenate(
        [t, jnp.zeros((BN_CONV, 48 - 34), F32)], axis=1)


def _mid_call(p1a, p1b, gcn_b, press_w, press_b, bn2_g, bn2_b, jk_w0):
    full = lambda shape: pl.BlockSpec(shape, lambda i: (0,) * len(shape))
    emd_full, tab2 = pl.pallas_call(
        _mid_body,
        grid=(NPAD // BN_CONV,),
        in_specs=[pl.BlockSpec((2, BN_CONV, 96), lambda i: (0, i, 0)),
                  pl.BlockSpec((2, BN_CONV, 80), lambda i: (0, i, 0)),
                  full((3, 54)), full((1, 3, 3)), full((1, 1)),
                  full((1, 54)), full((1, 54)), full((54, 34))],
        out_specs=(pl.BlockSpec((BN_CONV, 54), lambda i: (i, 0)),
                   pl.BlockSpec((BN_CONV, 48), lambda i: (i, 0))),
        out_shape=(jax.ShapeDtypeStruct((NPAD, 54), F32),
                   jax.ShapeDtypeStruct((NPAD, 48), F32)),
    )(p1a, p1b, gcn_b, press_w, press_b, bn2_g, bn2_b, jk_w0)
    return emd_full[:N], tab2


def _jk_body(du, dn, dnp, p_ref, b_ref, w_ref, tab_ref, ssum_ref):
    h = jax.nn.relu(p_ref[0, :N, :du] + p_ref[1, :N, :du] + b_ref[...])
    ssum_ref[...] = jnp.sum(h, axis=0, keepdims=True)
    t = _dot(h, w_ref[...])
    if dnp > dn:
        t = jnp.concatenate([t, jnp.zeros((N, dnp - dn), F32)], axis=1)
    tab_ref[...] = jnp.concatenate([t, jnp.zeros((NPAD - N, dnp), F32)], 0)


def _jk_call(p, b, w, du, dn, dnp):
    return pl.pallas_call(
        functools.partial(_jk_body, du, dn, dnp),
        out_shape=(jax.ShapeDtypeStruct((NPAD, dnp), F32),
                   jax.ShapeDtypeStruct((1, du), F32)),
    )(p, b, w)


def _final_body(p_ref, b_ref, s1_ref, s2_ref, bng_ref, bnb_ref,
                w1_ref, b1_ref, w2_ref, b2_ref, out_ref):
    h = jax.nn.relu(p_ref[0, :N, :16] + p_ref[1, :N, :16] + b_ref[...])
    s3 = jnp.sum(h, axis=0, keepdims=True)
    zc = jnp.concatenate([s1_ref[...], s2_ref[...], s3], axis=1)
    z = zc * (bng_ref[...] * BN_SCALE) + bnb_ref[...]
    l1 = _dot(z, w1_ref[...]) + b1_ref[...]
    out_ref[...] = _dot(l1, w2_ref[...]) + b2_ref[...]


def _final_call(p, b, s1, s2, bn3_g, bn3_b, fc1_wt, fc1_b, fc2_wt, fc2_b):
    return pl.pallas_call(
        _final_body,
        out_shape=jax.ShapeDtypeStruct((1, 10), F32),
    )(p, b, s1, s2, bn3_g, bn3_b, fc1_wt, fc1_b, fc2_wt, fc2_b)


def kernel(x, edge_weight, conv_w, conv_b, gcn_w, gcn_b, press_w, press_b,
           jk_w0, jk_b0, jk_w1, jk_b1, jk_w2, jk_b2, fc1_w, fc1_b,
           fc2_w, fc2_b, bn1_g, bn1_b, bn2_g, bn2_b, bn3_g, bn3_b,
           edge_index):
    src = jnp.concatenate(
        [edge_index[0].astype(jnp.int32), jnp.zeros((EPAD - E,), jnp.int32)])
    dst = jnp.concatenate(
        [edge_index[1].astype(jnp.int32), jnp.zeros((EPAD - E,), jnp.int32)])
    ew = jnp.concatenate(
        [edge_weight.astype(F32), jnp.zeros((EPAD - E,), F32)])

    tab1a, tab1b = _conv_call(x, conv_w, conv_b, bn1_g.reshape(1, 96),
                              bn1_b.reshape(1, 96), gcn_w)
    p1a = _seg_pass(tab1a, src, dst, ew, 96)
    p1b = _seg_pass(tab1b, src, dst, ew, 80)
    emd, tab2 = _mid_call(p1a, p1b, gcn_b, press_w, press_b.reshape(1, 1),
                          bn2_g.reshape(1, 54), bn2_b.reshape(1, 54), jk_w0)
    p2 = _seg_pass(tab2, src, dst, ew, 48)
    tab3, s1 = _jk_call(p2, jk_b0.reshape(1, 34), jk_w1, 34, 25, 32)
    p3 = _seg_pass(tab3, src, dst, ew, 32)
    tab4, s2 = _jk_call(p3, jk_b1.reshape(1, 25), jk_w2, 25, 16, 16)
    p4 = _seg_pass(tab4, src, dst, ew, 16)
    logits = _final_call(p4, jk_b2.reshape(1, 16), s1, s2,
                         bn3_g.reshape(1, 75), bn3_b.reshape(1, 75),
                         fc1_w.T, fc1_b.reshape(1, 120),
                         fc2_w.T, fc2_b.reshape(1, 10))
    return logits, emd


# speedup vs baseline: 7.7321x; 1.5264x over previous
import functools

import jax
import jax.numpy as jnp
from jax import lax
from jax.experimental import pallas as pl
from jax.experimental.pallas import tpu as pltpu
from jax.experimental.pallas import tpu_sc as plsc

N = 10000
NPAD = 10240
E = 320000
NCORES = 2
NSUB = 16
NW = NCORES * NSUB
CHUNK = 128
NCPT32 = 80
NCPT16 = 160
EPAD = NW * CHUNK * NCPT32
BN_SCALE = float(1.0 / (1.0 + 1e-5) ** 0.5)
F32 = jnp.float32


def _shift_rows(a, off, rows):
    if off == 0:
        return a
    z = jnp.zeros((abs(off), a.shape[1]), a.dtype)
    if off > 0:
        return jnp.concatenate([a[off:], z], axis=0)
    return jnp.concatenate([z, a[:off]], axis=0)


def _shift_cols(a, off):
    if off == 0:
        return a
    z = jnp.zeros((a.shape[0], abs(off)), a.dtype)
    if off > 0:
        return jnp.concatenate([a[:, off:], z], axis=1)
    return jnp.concatenate([z, a[:, :off]], axis=1)


def _dot(a, b):
    return jnp.dot(a, b, preferred_element_type=F32,
                   precision=jax.lax.Precision.HIGHEST)


BN_CONV = 2048
PXROWS = 3 + N + (NPAD + 6 - N - 3)


def _conv_body(px_ref, cw_ref, cb_ref, bng_ref, bnb_ref, gw_ref, taba_ref):
    i = pl.program_id(0)
    base = i * BN_CONV
    xs = {off: px_ref[pl.ds(base + 3 + off, BN_CONV), :]
          for off in range(-3, 4)}
    h0 = []
    for b in range(3):
        d = b + 1
        y = jnp.broadcast_to(cb_ref[b][None, :], (BN_CONV, 96)).astype(F32)
        for k in range(3):
            y = y + _dot(xs[(k - 1) * d], cw_ref[b, :, :, k].T)
        h0.append(y)
    scale = bng_ref[...] * BN_SCALE
    parts = []
    for b in range(3):
        h1 = (jax.nn.relu(h0[b]) + h0[(b + 1) % 3]) * scale + bnb_ref[...]
        parts.append(_dot(h1, gw_ref[b]))
    tab = jnp.concatenate(parts, axis=1)
    taba_ref[0] = tab[:, :96]
    taba_ref[1] = jnp.concatenate(
        [tab[:, 96:], jnp.zeros((BN_CONV, 96 - 66), F32)], axis=1)


def _conv_call(x, conv_w, conv_b, bn1_g, bn1_b, gcn_w):
    px = jnp.concatenate(
        [jnp.zeros((3, 128), F32), x, jnp.zeros((PXROWS - 3 - N, 128), F32)])
    full = lambda shape: pl.BlockSpec(shape, lambda i: (0,) * len(shape))
    return pl.pallas_call(
        _conv_body,
        grid=(NPAD // BN_CONV,),
        in_specs=[full((PXROWS, 128)), full((3, 96, 128, 3)), full((3, 96)),
                  full((1, 96)), full((1, 96)), full((3, 96, 54))],
        out_specs=pl.BlockSpec((2, BN_CONV, 96), lambda i: (0, i, 0)),
        out_shape=jax.ShapeDtypeStruct((2, NPAD, 96), F32),
    )(px, conv_w, conv_b, bn1_g, bn1_b, gcn_w)


_GDN = lax.GatherDimensionNumbers(
    offset_dims=(), collapsed_slice_dims=(0,), start_index_map=(0,))


def _bcast_lane(vec, i):
    idx = jnp.full((16, 1), i, jnp.int32)
    return lax.gather(vec, idx, _GDN, slice_sizes=(1,),
                      mode=lax.GatherScatterMode.PROMISE_IN_BOUNDS)


def _scale_rows(rows_ref, ew_get, nj):
    def group_body(g, carry):
        ewg = ew_get(g)
        for i in range(16):
            w = _bcast_lane(ewg, i)
            e = g * 16 + i
            for j in range(nj):
                rows_ref[e, pl.ds(16 * j, 16)] = (
                    rows_ref[e, pl.ds(16 * j, 16)] * w)
        return carry

    lax.fori_loop(0, CHUNK // 16, group_body, 0)


def _pass_body(dpad, ncpt, tab_h, packed_h, ew_h, out_h,
               idx_v, ew_v, rows0, rows1, acc, sem0, sem1):
    c = lax.axis_index("c")
    s = lax.axis_index("s")
    nj = dpad // 16
    tile = c * NSUB + s
    row_bufs = (rows0, rows1)
    sems = (sem0, sem1)

    pltpu.sync_copy(packed_h.at[pl.ds(tile * ncpt, ncpt)], idx_v)
    pltpu.sync_copy(ew_h.at[pl.ds(tile * ncpt, ncpt)], ew_v)

    def zero_row(r, carry):
        for j in range(nj):
            rows0[r, pl.ds(16 * j, 16)] = jnp.zeros((16,), F32)
        return carry

    lax.fori_loop(0, CHUNK, zero_row, 0)
    for q in range(NPAD // CHUNK // NSUB):
        blk = s * (NPAD // CHUNK // NSUB) + q
        pltpu.sync_copy(rows0, acc.at[pl.ds(blk * CHUNK, CHUNK)])
    plsc.subcore_barrier()

    def start_gather(ci, b):
        pltpu.async_copy(tab_h.at[idx_v.at[ci, 0]], row_bufs[b], sems[b])

    def wait_gather(ci, b):
        pltpu.make_async_copy(tab_h.at[idx_v.at[ci, 0]],
                              row_bufs[b], sems[b]).wait()

    start_gather(0, 0)

    def pair_body(kk, carry):
        for b in range(2):
            k = 2 * kk + b
            knext = lax.rem(k + 1, ncpt)
            start_gather(knext, 1 - b)
            wait_gather(k, b)
            _scale_rows(row_bufs[b],
                        lambda g: ew_v[k, pl.ds(g * 16, 16)], nj)
            pltpu.sync_copy(row_bufs[b], acc.at[idx_v.at[k, 1]], add=True)
        return carry

    lax.fori_loop(0, ncpt // 2, pair_body, 0)
    wait_gather(0, 0)
    plsc.subcore_barrier()

    rows_per = NPAD // NSUB
    pltpu.sync_copy(acc.at[pl.ds(s * rows_per, rows_per)],
                    out_h.at[c, pl.ds(s * rows_per, rows_per)])


def _pass1_body(tab_h, packed_h, ew_h, out_h,
                idx0, idx1, ewb0, ewb1, rows0, rows1, acc, sem0, sem1):
    c = lax.axis_index("c")
    s = lax.axis_index("s")
    nj = 96 // 16
    ncpt = NCPT16
    tile = s
    off = c * NPAD
    idx_bufs = (idx0, idx1)
    ew_bufs = (ewb0, ewb1)
    row_bufs = (rows0, rows1)
    sems = (sem0, sem1)

    def zero_row(r, carry):
        for j in range(nj):
            rows0[r, pl.ds(16 * j, 16)] = jnp.zeros((16,), F32)
        return carry

    lax.fori_loop(0, CHUNK, zero_row, 0)
    for q in range(NPAD // CHUNK // NSUB):
        blk = s * (NPAD // CHUNK // NSUB) + q
        pltpu.sync_copy(rows0, acc.at[pl.ds(blk * CHUNK, CHUNK)])
    plsc.subcore_barrier()

    def load_chunk(ci, b):
        pltpu.sync_copy(packed_h.at[tile * ncpt + ci], idx_bufs[b])
        pltpu.sync_copy(ew_h.at[tile * ncpt + ci], ew_bufs[b])
        for j in range(CHUNK // 16):
            idx_bufs[b][0, pl.ds(16 * j, 16)] = (
                idx_bufs[b][0, pl.ds(16 * j, 16)] + off)

    def start_gather(b):
        pltpu.async_copy(tab_h.at[idx_bufs[b].at[0]], row_bufs[b], sems[b])

    def wait_gather(b):
        pltpu.make_async_copy(tab_h.at[idx_bufs[b].at[0]],
                              row_bufs[b], sems[b]).wait()

    load_chunk(0, 0)
    start_gather(0)

    def pair_body(kk, carry):
        for b in range(2):
            k = 2 * kk + b
            load_chunk(lax.rem(k + 1, ncpt), 1 - b)
            start_gather(1 - b)
            wait_gather(b)
            _scale_rows(row_bufs[b],
                        lambda g: ew_bufs[b][pl.ds(g * 16, 16)], nj)
            pltpu.sync_copy(row_bufs[b], acc.at[idx_bufs[b].at[1]], add=True)
        return carry

    lax.fori_loop(0, ncpt // 2, pair_body, 0)
    wait_gather(0)
    plsc.subcore_barrier()

    rows_per = NPAD // NSUB
    pltpu.sync_copy(acc.at[pl.ds(s * rows_per, rows_per)],
                    out_h.at[c, pl.ds(s * rows_per, rows_per)])


def _seg_pass(table, packed, ew2, dpad, stage1):
    mesh = plsc.VectorSubcoreMesh(core_axis_name="c", subcore_axis_name="s")
    if stage1:
        body = _pass1_body
        scratch = [
            pltpu.VMEM((2, CHUNK), jnp.int32),
            pltpu.VMEM((2, CHUNK), jnp.int32),
            pltpu.VMEM((CHUNK,), F32),
            pltpu.VMEM((CHUNK,), F32),
            pltpu.VMEM((CHUNK, dpad), F32),
            pltpu.VMEM((CHUNK, dpad), F32),
            pltpu.VMEM_SHARED((NPAD, dpad), F32),
            pltpu.SemaphoreType.DMA,
            pltpu.SemaphoreType.DMA,
        ]
    else:
        body = functools.partial(_pass_body, dpad, NCPT32)
        scratch = [
            pltpu.VMEM((NCPT32, 2, CHUNK), jnp.int32),
            pltpu.VMEM((NCPT32, CHUNK), F32),
            pltpu.VMEM((CHUNK, dpad), F32),
            pltpu.VMEM((CHUNK, dpad), F32),
            pltpu.VMEM_SHARED((NPAD, dpad), F32),
            pltpu.SemaphoreType.DMA,
            pltpu.SemaphoreType.DMA,
        ]
    f = pl.kernel(
        body,
        mesh=mesh,
        compiler_params=pltpu.CompilerParams(use_tc_tiling_on_sc=False),
        out_type=jax.ShapeDtypeStruct((2, NPAD, dpad), F32),
        scratch_types=scratch,
    )
    return f(table, packed, ew2)


def _mid_body(pa_ref, gb_ref, pw_ref, pb2_ref, bng_ref, bnb_ref,
              w0_ref, emd_ref, tab_ref):
    agg = jnp.concatenate([pa_ref[0], pa_ref[1, :, :66]], axis=1)
    h2 = [jax.nn.relu(agg[:, 54 * i:54 * (i + 1)] + gb_ref[i][None, :])
          for i in range(3)]
    pressed = jnp.full((BN_CONV, 54), pb2_ref[0, 0], F32)
    for cidx in range(3):
        for k in range(3):
            pressed = pressed + pw_ref[0, cidx, k] * _shift_cols(h2[cidx], k - 1)
    emd = pressed * (bng_ref[...] * BN_SCALE) + bnb_ref[...]
    emd_ref[...] = emd
    t = _dot(emd, w0_ref[...])
    tab_ref[...] = jnp.concatenate(
        [t, jnp.zeros((BN_CONV, 48 - 34), F32)], axis=1)


def _mid_call(p1, gcn_b, press_w, press_b, bn2_g, bn2_b, jk_w0):
    full = lambda shape: pl.BlockSpec(shape, lambda i: (0,) * len(shape))
    emd_full, tab2 = pl.pallas_call(
        _mid_body,
        grid=(NPAD // BN_CONV,),
        in_specs=[pl.BlockSpec((2, BN_CONV, 96), lambda i: (0, i, 0)),
                  full((3, 54)), full((1, 3, 3)), full((1, 1)),
                  full((1, 54)), full((1, 54)), full((54, 34))],
        out_specs=(pl.BlockSpec((BN_CONV, 54), lambda i: (i, 0)),
                   pl.BlockSpec((BN_CONV, 48), lambda i: (i, 0))),
        out_shape=(jax.ShapeDtypeStruct((NPAD, 54), F32),
                   jax.ShapeDtypeStruct((NPAD, 48), F32)),
    )(p1, gcn_b, press_w, press_b, bn2_g, bn2_b, jk_w0)
    return emd_full[:N], tab2


def _jk_body(du, dn, dnp, p_ref, b_ref, w_ref, tab_ref, ssum_ref):
    h = jax.nn.relu(p_ref[0, :N, :du] + p_ref[1, :N, :du] + b_ref[...])
    ssum_ref[...] = jnp.sum(h, axis=0, keepdims=True)
    t = _dot(h, w_ref[...])
    if dnp > dn:
        t = jnp.concatenate([t, jnp.zeros((N, dnp - dn), F32)], axis=1)
    tab_ref[...] = jnp.concatenate([t, jnp.zeros((NPAD - N, dnp), F32)], 0)


def _jk_call(p, b, w, du, dn, dnp):
    return pl.pallas_call(
        functools.partial(_jk_body, du, dn, dnp),
        out_shape=(jax.ShapeDtypeStruct((NPAD, dnp), F32),
                   jax.ShapeDtypeStruct((1, du), F32)),
    )(p, b, w)


def _final_body(p_ref, b_ref, s1_ref, s2_ref, bng_ref, bnb_ref,
                w1_ref, b1_ref, w2_ref, b2_ref, out_ref):
    h = jax.nn.relu(p_ref[0, :N, :16] + p_ref[1, :N, :16] + b_ref[...])
    s3 = jnp.sum(h, axis=0, keepdims=True)
    zc = jnp.concatenate([s1_ref[...], s2_ref[...], s3], axis=1)
    z = zc * (bng_ref[...] * BN_SCALE) + bnb_ref[...]
    l1 = _dot(z, w1_ref[...]) + b1_ref[...]
    out_ref[...] = _dot(l1, w2_ref[...]) + b2_ref[...]


def _final_call(p, b, s1, s2, bn3_g, bn3_b, fc1_wt, fc1_b, fc2_wt, fc2_b):
    return pl.pallas_call(
        _final_body,
        out_shape=jax.ShapeDtypeStruct((1, 10), F32),
    )(p, b, s1, s2, bn3_g, bn3_b, fc1_wt, fc1_b, fc2_wt, fc2_b)


def kernel(x, edge_weight, conv_w, conv_b, gcn_w, gcn_b, press_w, press_b,
           jk_w0, jk_b0, jk_w1, jk_b1, jk_w2, jk_b2, fc1_w, fc1_b,
           fc2_w, fc2_b, bn1_g, bn1_b, bn2_g, bn2_b, bn3_g, bn3_b,
           edge_index):
    src = jnp.concatenate(
        [edge_index[0].astype(jnp.int32), jnp.zeros((EPAD - E,), jnp.int32)])
    dst = jnp.concatenate(
        [edge_index[1].astype(jnp.int32), jnp.zeros((EPAD - E,), jnp.int32)])
    ew = jnp.concatenate(
        [edge_weight.astype(F32), jnp.zeros((EPAD - E,), F32)])
    packed = jnp.stack([src.reshape(-1, CHUNK), dst.reshape(-1, CHUNK)],
                       axis=1)
    ew2 = ew.reshape(-1, CHUNK)

    tab1 = _conv_call(x, conv_w, conv_b, bn1_g.reshape(1, 96),
                      bn1_b.reshape(1, 96), gcn_w)
    p1 = _seg_pass(tab1.reshape(2 * NPAD, 96), packed, ew2, 96, True)
    emd, tab2 = _mid_call(p1, gcn_b, press_w, press_b.reshape(1, 1),
                          bn2_g.reshape(1, 54), bn2_b.reshape(1, 54), jk_w0)
    p2 = _seg_pass(tab2, packed, ew2, 48, False)
    tab3, s1 = _jk_call(p2, jk_b0.reshape(1, 34), jk_w1, 34, 25, 32)
    p3 = _seg_pass(tab3, packed, ew2, 32, False)
    tab4, s2 = _jk_call(p3, jk_b1.reshape(1, 25), jk_w2, 25, 16, 16)
    p4 = _seg_pass(tab4, packed, ew2, 16, False)
    logits = _final_call(p4, jk_b2.reshape(1, 16), s1, s2,
                         bn3_g.reshape(1, 75), bn3_b.reshape(1, 75),
                         fc1_w.T, fc1_b.reshape(1, 120),
                         fc2_w.T, fc2_b.reshape(1, 10))
    return logits, emd
